# baseline (speedup 1.0000x reference)
import functools

import jax
import jax.numpy as jnp
from jax.experimental import pallas as pl
from jax.experimental.pallas import tpu as pltpu

_TOPK = 3


def _body(qrr_ref, qii_ref, qcat_ref, pm_r_ref, pm_i_ref,
          WaG_ref, baG_ref, Wg_ref, bg_ref,
          gr_ref, br_ref, gi_ref, bi_ref,
          read_ref, next_ref, ent_ref, *, total_b):
    pm_r = pm_r_ref[...]
    pm_i = pm_i_ref[...]
    bB, SH, L = pm_r.shape
    S = 2 * SH
    D = L // 2

    lane = jax.lax.broadcasted_iota(jnp.int32, (1, 1, L), 2)
    lo = lane < D
    sgn = jnp.where(lo, 1.0, -1.0)

    qrr = qrr_ref[...][:, None, :]
    qii = qii_ref[...][:, None, :]

    p = pm_r * qrr + pm_i * qii
    f = jnp.sum(p, axis=-1)
    g = jnp.sum(p * sgn, axis=-1)
    se = (f + g) * 0.5
    so = (f - g) * 0.5
    sim = jnp.concatenate([se, so], axis=-1)
    sim = sim - jnp.max(sim, axis=-1, keepdims=True)
    es = jnp.exp(sim)
    attn = es / jnp.sum(es, axis=-1, keepdims=True)

    a_ev = attn[:, :SH][:, :, None]
    a_od = attn[:, SH:][:, :, None]
    A = jnp.where(lo, a_ev, a_od)
    rr = jnp.sum(A * pm_r, axis=1)
    ri = jnp.sum(A * pm_i, axis=1)
    read_ref[0] = rr[:, :D] + rr[:, D:]
    read_ref[1] = ri[:, :D] + ri[:, D:]

    qcat = qcat_ref[...]
    glogit = jnp.sum(qcat * Wg_ref[...], axis=-1, keepdims=True) + bg_ref[0, 0]
    gate = jax.nn.sigmoid(glogit)

    logits = jax.lax.dot_general(qcat, WaG_ref[...], (((1,), (0,)), ((), ())),
                                 preferred_element_type=jnp.float32)
    logits = logits + baG_ref[...]
    logits = logits - jnp.max(logits, axis=-1, keepdims=True)
    ew = jnp.exp(logits)
    w = ew / jnp.sum(ew, axis=-1, keepdims=True)

    ent_rows = -jnp.sum(w * jnp.log(w + 1e-10), axis=-1, keepdims=True)
    ent_part = jnp.sum(ent_rows) / (total_b * 8.0 * 128.0)
    ent_ref[...] = jnp.full((8, 128), ent_part, dtype=jnp.float32)

    iota = jax.lax.broadcasted_iota(jnp.int32, (bB, S), 1)
    remaining = w
    keep = jnp.zeros(w.shape, dtype=jnp.bool_)
    for _ in range(_TOPK):
        mx = jnp.max(remaining, axis=-1, keepdims=True)
        first = jnp.min(jnp.where(remaining == mx, iota, S), axis=-1,
                        keepdims=True)
        onehot = iota == first
        keep = jnp.logical_or(keep, onehot)
        remaining = jnp.where(onehot, -1.0, remaining)
    sparse = jnp.where(keep, w, 0.0)
    sparse = sparse / (jnp.sum(sparse, axis=-1, keepdims=True) + 1e-6)
    eff = gate * sparse

    e_ev = eff[:, :SH][:, :, None]
    e_od = eff[:, SH:][:, :, None]
    E = jnp.where(lo, e_ev, e_od)

    def update_norm(pm, qt, gam_ref, bet_ref):
        y = jnp.tanh(pm + E * (qt - pm))
        f2 = jnp.sum(y, axis=-1, keepdims=True)
        g2 = jnp.sum(y * sgn, axis=-1, keepdims=True)
        m_lo = (f2 + g2) * (0.5 / D)
        m_hi = (f2 - g2) * (0.5 / D)
        mub = jnp.where(lo, m_lo, m_hi)
        d = y - mub
        dd = d * d
        vf = jnp.sum(dd, axis=-1, keepdims=True)
        vg = jnp.sum(dd * sgn, axis=-1, keepdims=True)
        v_lo = (vf + vg) * (0.5 / D)
        v_hi = (vf - vg) * (0.5 / D)
        varb = jnp.where(lo, v_lo, v_hi)
        return (d * jax.lax.rsqrt(varb + 1e-6) * gam_ref[...][:, None, :]
                + bet_ref[...][:, None, :])

    next_ref[0] = update_norm(pm_r, qrr, gr_ref, br_ref)
    next_ref[1] = update_norm(pm_i, qii, gi_ref, bi_ref)


def kernel(gw_state_real, gw_state_imag, prev_mem_real, prev_mem_imag,
           Wg, bg, Wa, ba, gamma_r, beta_r, gamma_i, beta_i):
    B, S, D = prev_mem_real.shape
    SH, L = S // 2, 2 * D
    bB = 64
    grid = (B // bB,)

    pm2_r = prev_mem_real.reshape(B, SH, L)
    pm2_i = prev_mem_imag.reshape(B, SH, L)
    qrr = jnp.concatenate([gw_state_real, gw_state_real], axis=-1)
    qii = jnp.concatenate([gw_state_imag, gw_state_imag], axis=-1)
    qcat = jnp.concatenate([gw_state_real, gw_state_imag], axis=-1)
    WaG = jnp.concatenate([Wa[0::2], Wa[1::2]], axis=0).T
    baG = jnp.concatenate([ba[0::2], ba[1::2]]).reshape(1, S)
    bg2 = bg.reshape(1, 1)
    gr2 = jnp.concatenate([gamma_r, gamma_r]).reshape(1, L)
    br2 = jnp.concatenate([beta_r, beta_r]).reshape(1, L)
    gi2 = jnp.concatenate([gamma_i, gamma_i]).reshape(1, L)
    bi2 = jnp.concatenate([beta_i, beta_i]).reshape(1, L)

    def row_map(i):
        return (i, 0)

    def mem_map(i):
        return (i, 0, 0)

    def const2(i):
        return (0, 0)

    read_out, next2, ent = pl.pallas_call(
        functools.partial(_body, total_b=float(B)),
        grid=grid,
        in_specs=[
            pl.BlockSpec((bB, L), row_map),
            pl.BlockSpec((bB, L), row_map),
            pl.BlockSpec((bB, L), row_map),
            pl.BlockSpec((bB, SH, L), mem_map),
            pl.BlockSpec((bB, SH, L), mem_map),
            pl.BlockSpec((L, S), const2),
            pl.BlockSpec((1, S), const2),
            pl.BlockSpec((1, L), const2),
            pl.BlockSpec((1, 1), const2),
            pl.BlockSpec((1, L), const2),
            pl.BlockSpec((1, L), const2),
            pl.BlockSpec((1, L), const2),
            pl.BlockSpec((1, L), const2),
        ],
        out_specs=[
            pl.BlockSpec((2, bB, D), lambda i: (0, i, 0)),
            pl.BlockSpec((2, bB, SH, L), lambda i: (0, i, 0, 0)),
            pl.BlockSpec((8, 128), lambda i: (i, 0)),
        ],
        out_shape=[
            jax.ShapeDtypeStruct((2, B, D), jnp.float32),
            jax.ShapeDtypeStruct((2, B, SH, L), jnp.float32),
            jax.ShapeDtypeStruct((grid[0] * 8, 128), jnp.float32),
        ],
        compiler_params=pltpu.CompilerParams(
            dimension_semantics=("parallel",),
        ),
    )(qrr, qii, qcat, pm2_r, pm2_i, WaG, baG, Wg, bg2, gr2, br2, gi2, bi2)

    return (read_out, next2.reshape(2, B, S, D), jnp.sum(ent))


# speedup vs baseline: 1.0632x; 1.0632x over previous
import functools

import jax
import jax.numpy as jnp
from jax.experimental import pallas as pl
from jax.experimental.pallas import tpu as pltpu

_TOPK = 3


def _body(q_r_ref, q_i_ref, qcat_ref, pm_r_ref, pm_i_ref,
          WaT_ref, ba_ref, Wg_ref, bg_ref,
          gr_ref, br_ref, gi_ref, bi_ref,
          read_ref, next_ref, ent_ref, *, total_b):
    pm_r = pm_r_ref[...]
    pm_i = pm_i_ref[...]
    bB, S, D = pm_r.shape

    q_r = q_r_ref[...][:, None, :]
    q_i = q_i_ref[...][:, None, :]

    sim = jnp.sum(pm_r * q_r + pm_i * q_i, axis=-1)
    sim = sim - jnp.max(sim, axis=-1, keepdims=True)
    es = jnp.exp(sim)
    attn = es / jnp.sum(es, axis=-1, keepdims=True)
    a3 = attn[:, :, None]
    read_ref[0] = jnp.sum(pm_r * a3, axis=1)
    read_ref[1] = jnp.sum(pm_i * a3, axis=1)

    qcat = qcat_ref[...]
    glogit = jnp.sum(qcat * Wg_ref[...], axis=-1, keepdims=True) + bg_ref[0, 0]
    gate = jax.nn.sigmoid(glogit)

    logits = jax.lax.dot_general(qcat, WaT_ref[...], (((1,), (0,)), ((), ())),
                                 preferred_element_type=jnp.float32)
    logits = logits + ba_ref[...]
    logits = logits - jnp.max(logits, axis=-1, keepdims=True)
    ew = jnp.exp(logits)
    w = ew / jnp.sum(ew, axis=-1, keepdims=True)

    ent_rows = -jnp.sum(w * jnp.log(w + 1e-10), axis=-1, keepdims=True)
    ent_part = jnp.sum(ent_rows) / (total_b * 8.0 * 128.0)
    ent_ref[...] = jnp.full((8, 128), ent_part, dtype=jnp.float32)

    iota = jax.lax.broadcasted_iota(jnp.int32, (bB, S), 1)
    remaining = w
    keep = jnp.zeros(w.shape, dtype=jnp.bool_)
    for _ in range(_TOPK):
        mx = jnp.max(remaining, axis=-1, keepdims=True)
        first = jnp.min(jnp.where(remaining == mx, iota, S), axis=-1,
                        keepdims=True)
        onehot = iota == first
        keep = jnp.logical_or(keep, onehot)
        remaining = jnp.where(onehot, -1.0, remaining)
    sparse = jnp.where(keep, w, 0.0)
    sparse = sparse / (jnp.sum(sparse, axis=-1, keepdims=True) + 1e-6)
    eff = gate * sparse
    e3 = eff[:, :, None]

    def update_norm(pm, q, gam_ref, bet_ref):
        y = jnp.tanh(pm + e3 * (q - pm))
        mu = jnp.mean(y, axis=-1, keepdims=True)
        d = y - mu
        var = jnp.mean(d * d, axis=-1, keepdims=True)
        return (d * jax.lax.rsqrt(var + 1e-6) * gam_ref[...][:, None, :]
                + bet_ref[...][:, None, :])

    next_ref[0] = update_norm(pm_r, q_r, gr_ref, br_ref)
    next_ref[1] = update_norm(pm_i, q_i, gi_ref, bi_ref)


def kernel(gw_state_real, gw_state_imag, prev_mem_real, prev_mem_imag,
           Wg, bg, Wa, ba, gamma_r, beta_r, gamma_i, beta_i):
    B, S, D = prev_mem_real.shape
    bB = 64
    grid = (B // bB,)

    qcat = jnp.concatenate([gw_state_real, gw_state_imag], axis=-1)
    WaT = Wa.T
    ba2 = ba.reshape(1, S)
    bg2 = bg.reshape(1, 1)
    gr2 = gamma_r.reshape(1, D)
    br2 = beta_r.reshape(1, D)
    gi2 = gamma_i.reshape(1, D)
    bi2 = beta_i.reshape(1, D)

    def row_map(i):
        return (i, 0)

    def mem_map(i):
        return (i, 0, 0)

    def const2(i):
        return (0, 0)

    read_out, next_mem, ent = pl.pallas_call(
        functools.partial(_body, total_b=float(B)),
        grid=grid,
        in_specs=[
            pl.BlockSpec((bB, D), row_map),
            pl.BlockSpec((bB, D), row_map),
            pl.BlockSpec((bB, 2 * D), row_map),
            pl.BlockSpec((bB, S, D), mem_map),
            pl.BlockSpec((bB, S, D), mem_map),
            pl.BlockSpec((2 * D, S), const2),
            pl.BlockSpec((1, S), const2),
            pl.BlockSpec((1, 2 * D), const2),
            pl.BlockSpec((1, 1), const2),
            pl.BlockSpec((1, D), const2),
            pl.BlockSpec((1, D), const2),
            pl.BlockSpec((1, D), const2),
            pl.BlockSpec((1, D), const2),
        ],
        out_specs=[
            pl.BlockSpec((2, bB, D), lambda i: (0, i, 0)),
            pl.BlockSpec((2, bB, S, D), lambda i: (0, i, 0, 0)),
            pl.BlockSpec((8, 128), lambda i: (i, 0)),
        ],
        out_shape=[
            jax.ShapeDtypeStruct((2, B, D), jnp.float32),
            jax.ShapeDtypeStruct((2, B, S, D), jnp.float32),
            jax.ShapeDtypeStruct((grid[0] * 8, 128), jnp.float32),
        ],
        compiler_params=pltpu.CompilerParams(
            dimension_semantics=("parallel",),
        ),
    )(gw_state_real, gw_state_imag, qcat, prev_mem_real, prev_mem_imag,
      WaT, ba2, Wg, bg2, gr2, br2, gi2, bi2)

    return (read_out, next_mem, jnp.sum(ent))
